# baseline (speedup 1.0000x reference)
import functools

import jax
import jax.numpy as jnp
from jax import lax
from jax.experimental import pallas as pl
from jax.experimental.pallas import tpu as pltpu
from jax.experimental.pallas import tpu_sc as plsc

B = 16384
D = 64
H = 64

NC, NS = 2, 16
NW = NC * NS
BPW = B // NW
CH = 128
NCH = BPW // CH

_sc_mesh = plsc.VectorSubcoreMesh(
    core_axis_name="c", subcore_axis_name="s", num_cores=NC, num_subcores=NS
)


@functools.partial(
    pl.kernel,
    out_type=(
        jax.ShapeDtypeStruct((B, D), jnp.float32),
        jax.ShapeDtypeStruct((B, D), jnp.float32),
    ),
    mesh=_sc_mesh,
    scratch_types=[
        pltpu.VMEM((NCH, CH), jnp.int32),
        pltpu.VMEM((NCH, CH), jnp.int32),
        pltpu.VMEM((BPW, D), jnp.float32),
        pltpu.VMEM((BPW, D), jnp.float32),
        pltpu.SemaphoreType.DMA,
    ],
    compiler_params=pltpu.CompilerParams(use_tc_tiling_on_sc=False),
)
def _sc_gather(uid_hbm, qid_hbm, ut_hbm, qt_hbm, uout_hbm, qout_hbm,
               uidx_v, qidx_v, urows_v, qrows_v, sem):
    wid = lax.axis_index("s") * NC + lax.axis_index("c")
    base = wid * BPW
    pltpu.sync_copy(uid_hbm.at[wid], uidx_v)
    pltpu.sync_copy(qid_hbm.at[wid], qidx_v)
    copies = []
    for j in range(NCH):
        copies.append(pltpu.async_copy(
            ut_hbm.at[uidx_v.at[j]], urows_v.at[pl.ds(j * CH, CH)], sem))
        copies.append(pltpu.async_copy(
            qt_hbm.at[qidx_v.at[j]], qrows_v.at[pl.ds(j * CH, CH)], sem))
    for c in copies:
        c.wait()
    pltpu.sync_copy(urows_v, uout_hbm.at[pl.ds(base, BPW)])
    pltpu.sync_copy(qrows_v, qout_hbm.at[pl.ds(base, BPW)])


BLK = 2048


def _mlp_body(u_ref, q_ref, w1u_ref, w1q_ref, b1_ref, w2_ref, b2_ref, o_ref):
    h = jnp.dot(u_ref[...], w1u_ref[...], preferred_element_type=jnp.float32)
    h = h + jnp.dot(q_ref[...], w1q_ref[...], preferred_element_type=jnp.float32)
    h = jnp.maximum(h + b1_ref[...], 0.0)
    o = jnp.sum(h * w2_ref[...], axis=1, keepdims=True) + b2_ref[...]
    o_ref[...] = jax.nn.sigmoid(o)


def _mlp(u, q, w1u, w1q, b1, w2t, b2):
    grid = (B // BLK,)
    return pl.pallas_call(
        _mlp_body,
        grid=grid,
        in_specs=[
            pl.BlockSpec((BLK, D), lambda i: (i, 0)),
            pl.BlockSpec((BLK, D), lambda i: (i, 0)),
            pl.BlockSpec((D, H), lambda i: (0, 0)),
            pl.BlockSpec((D, H), lambda i: (0, 0)),
            pl.BlockSpec((1, H), lambda i: (0, 0)),
            pl.BlockSpec((1, H), lambda i: (0, 0)),
            pl.BlockSpec((1, 1), lambda i: (0, 0)),
        ],
        out_specs=pl.BlockSpec((BLK, 1), lambda i: (i, 0)),
        out_shape=jax.ShapeDtypeStruct((B, 1), jnp.float32),
    )(u, q, w1u, w1q, b1, w2t, b2)


def kernel(user_id, question_id, user_table, question_table, W1, b1, W2, b2):
    uid = user_id.astype(jnp.int32).reshape(NW, NCH, CH)
    qid = question_id.astype(jnp.int32).reshape(NW, NCH, CH)
    u, q = _sc_gather(uid, qid, user_table, question_table)
    w1u = W1[:D]
    w1q = W1[D:]
    b1r = b1.reshape(1, H)
    w2t = W2.reshape(1, H)
    b2r = b2.reshape(1, 1)
    return _mlp(u, q, w1u, w1q, b1r, w2t, b2r)


# speedup vs baseline: 1.6443x; 1.6443x over previous
import functools

import jax
import jax.numpy as jnp
from jax import lax
from jax.experimental import pallas as pl
from jax.experimental.pallas import tpu as pltpu
from jax.experimental.pallas import tpu_sc as plsc

B = 16384
D = 64
H = 64

NC, NS = 2, 16
NW = NC * NS
BPW = B // NW
CH = 256
NCH = BPW // CH

_sc_mesh = plsc.VectorSubcoreMesh(
    core_axis_name="c", subcore_axis_name="s", num_cores=NC, num_subcores=NS
)


@functools.partial(
    pl.kernel,
    out_type=(
        jax.ShapeDtypeStruct((B, D), jnp.float32),
        jax.ShapeDtypeStruct((B, D), jnp.float32),
    ),
    mesh=_sc_mesh,
    scratch_types=[
        pltpu.VMEM((BPW,), jnp.int32),
        pltpu.VMEM((BPW,), jnp.int32),
        pltpu.VMEM((CH, D), jnp.float32),
        pltpu.VMEM((CH, D), jnp.float32),
        pltpu.SemaphoreType.DMA,
    ],
)
def _sc_gather(uid_hbm, qid_hbm, ut_hbm, qt_hbm, uout_hbm, qout_hbm,
               uidx_v, qidx_v, urows_v, qrows_v, sem):
    wid = lax.axis_index("s") * NC + lax.axis_index("c")
    base = wid * BPW
    pltpu.sync_copy(uid_hbm.at[pl.ds(base, BPW)], uidx_v)
    pltpu.sync_copy(qid_hbm.at[pl.ds(base, BPW)], qidx_v)
    for h in range(NCH):
        off = h * CH

        def body(g, _, off=off):
            uvec = uidx_v[pl.ds(off + g * 16, 16)]
            qvec = qidx_v[pl.ds(off + g * 16, 16)]
            for k in range(16):
                pltpu.async_copy(ut_hbm.at[uvec[k]], urows_v.at[g * 16 + k], sem)
                pltpu.async_copy(qt_hbm.at[qvec[k]], qrows_v.at[g * 16 + k], sem)
            return 0

        lax.fori_loop(0, CH // 16, body, 0)
        pltpu.make_async_copy(ut_hbm.at[pl.ds(0, CH)], urows_v, sem).wait()
        pltpu.make_async_copy(qt_hbm.at[pl.ds(0, CH)], qrows_v, sem).wait()
        pltpu.sync_copy(urows_v, uout_hbm.at[pl.ds(base + off, CH)])
        pltpu.sync_copy(qrows_v, qout_hbm.at[pl.ds(base + off, CH)])


BLK = 2048


def _mlp_body(u_ref, q_ref, w1u_ref, w1q_ref, b1_ref, w2_ref, b2_ref, o_ref):
    h = jnp.dot(u_ref[...], w1u_ref[...], preferred_element_type=jnp.float32)
    h = h + jnp.dot(q_ref[...], w1q_ref[...], preferred_element_type=jnp.float32)
    h = jnp.maximum(h + b1_ref[...], 0.0)
    o = jnp.sum(h * w2_ref[...], axis=1, keepdims=True) + b2_ref[...]
    o_ref[...] = jax.nn.sigmoid(o)


def _mlp(u, q, w1u, w1q, b1, w2t, b2):
    grid = (B // BLK,)
    return pl.pallas_call(
        _mlp_body,
        grid=grid,
        in_specs=[
            pl.BlockSpec((BLK, D), lambda i: (i, 0)),
            pl.BlockSpec((BLK, D), lambda i: (i, 0)),
            pl.BlockSpec((D, H), lambda i: (0, 0)),
            pl.BlockSpec((D, H), lambda i: (0, 0)),
            pl.BlockSpec((1, H), lambda i: (0, 0)),
            pl.BlockSpec((1, H), lambda i: (0, 0)),
            pl.BlockSpec((1, 1), lambda i: (0, 0)),
        ],
        out_specs=pl.BlockSpec((BLK, 1), lambda i: (i, 0)),
        out_shape=jax.ShapeDtypeStruct((B, 1), jnp.float32),
    )(u, q, w1u, w1q, b1, w2t, b2)


def kernel(user_id, question_id, user_table, question_table, W1, b1, W2, b2):
    uid = user_id.astype(jnp.int32)
    qid = question_id.astype(jnp.int32)
    u, q = _sc_gather(uid, qid, user_table, question_table)
    w1u = W1[:D]
    w1q = W1[D:]
    b1r = b1.reshape(1, H)
    w2t = W2.reshape(1, H)
    b2r = b2.reshape(1, 1)
    return _mlp(u, q, w1u, w1q, b1r, w2t, b2r)


# speedup vs baseline: 1.7371x; 1.0564x over previous
import functools

import jax
import jax.numpy as jnp
from jax import lax
from jax.experimental import pallas as pl
from jax.experimental.pallas import tpu as pltpu
from jax.experimental.pallas import tpu_sc as plsc

B = 16384
D = 64
H = 64

NC, NS = 2, 16
NW = NC * NS

NU = 1000000
NQ = 100000
TAIL = 256
TU = NU - TAIL
TQ = NQ - TAIL
SLAB = 512
NS_U = 1953
NS_Q = 195
RING = 64

_sc_mesh = plsc.VectorSubcoreMesh(
    core_axis_name="c", subcore_axis_name="s", num_cores=NC, num_subcores=NS
)


def _phase(idv, tab_hbm, out_hbm, listv, posv, bufs, stage,
           sA, sB, sOut, s0, ns, tcut):
    lo_r = s0 * SLAB
    hi_r = jnp.minimum((s0 + ns) * SLAB, tcut)

    def dma_slab(t, buf_ix, sem):
        s = s0 + t
        pltpu.async_copy(
            tab_hbm.at[:, pl.ds(s * SLAB, SLAB)], bufs.at[buf_ix], sem)

    def drain_slab(buf_ix, sem):
        pltpu.make_async_copy(
            tab_hbm.at[:, pl.ds(0, SLAB)], bufs.at[buf_ix], sem).wait()

    @pl.when(ns > 0)
    def _():
        dma_slab(0, 0, sA)

    @pl.when(ns > 1)
    def _():
        dma_slab(1, 1, sB)

    def compact_body(g, cnt):
        vec = idv[pl.ds(g * 16, 16)]
        posvec = lax.iota(jnp.int32, 16) + g * 16
        m = (vec >= lo_r) & (vec < hi_r)
        plsc.store_compressed(listv.at[pl.ds(cnt, 16)], vec, mask=m)
        plsc.store_compressed(posv.at[pl.ds(cnt, 16)], posvec, mask=m)
        return cnt + plsc.all_reduce_population_count(m)[0]

    cnt = lax.fori_loop(0, B // 16, compact_body, jnp.int32(0))
    nv = (cnt + 15) >> 4

    f0 = lax.iota(jnp.int32, 16)

    def process_slab(t, buf_ix, n):
        s = s0 + t
        p16 = jnp.full((16,), buf_ix, jnp.int32)

        def vreg_body(v, n):
            uvec = listv[pl.ds(v * 16, 16)]
            pvec = posv[pl.ds(v * 16, 16)]
            in_list = (lax.iota(jnp.int32, 16) + v * 16) < cnt
            m = ((uvec >> 9) == s) & in_list
            mi = jnp.where(m, 1, 0)
            offc = jnp.clip(uvec - s * SLAB, 0, SLAB - 1)
            hits = plsc.all_reduce_population_count(m)[0]

            def extract(n):
                for k in range(16):
                    ck = mi[k] == 1

                    @pl.when(ck & (n + 0 >= RING))
                    def _():
                        pltpu.make_async_copy(
                            out_hbm.at[0], stage.at[0], sOut).wait()

                    @pl.when(ck)
                    def _(k=k, n=n):
                        slot = lax.rem(n, jnp.int32(RING))
                        o16 = jnp.full((16,), offc[k], jnp.int32)
                        for mm in range(4):
                            seg = plsc.load_gather(
                                bufs, [p16, f0 + 16 * mm, o16])
                            stage[slot, pl.ds(16 * mm, 16)] = seg
                        pltpu.async_copy(
                            stage.at[slot], out_hbm.at[pvec[k]], sOut)

                    n = n + mi[k]
                return n

            return lax.cond(hits > 0, extract, lambda n: n, n)

        return lax.fori_loop(0, nv, vreg_body, n)

    def pair_body(i, n):
        t0 = 2 * i
        t1 = 2 * i + 1

        @pl.when(t0 < ns)
        def _():
            drain_slab(0, sA)

        n = lax.cond(t0 < ns, lambda n: process_slab(t0, 0, n),
                     lambda n: n, n)

        @pl.when(t0 + 2 < ns)
        def _():
            dma_slab(t0 + 2, 0, sA)

        @pl.when(t1 < ns)
        def _():
            drain_slab(1, sB)

        n = lax.cond(t1 < ns, lambda n: process_slab(t1, 1, n),
                     lambda n: n, n)

        @pl.when(t1 + 2 < ns)
        def _():
            dma_slab(t1 + 2, 1, sB)
        return n

    n = lax.fori_loop(0, (ns + 1) >> 1, pair_body, jnp.int32(0))

    def drain_out(i, _):
        pltpu.make_async_copy(out_hbm.at[0], stage.at[0], sOut).wait()
        return 0

    lax.fori_loop(0, jnp.minimum(n, RING), drain_out, 0)


@functools.partial(
    pl.kernel,
    out_type=(
        jax.ShapeDtypeStruct((B, D), jnp.float32),
        jax.ShapeDtypeStruct((B, D), jnp.float32),
    ),
    mesh=_sc_mesh,
    scratch_types=[
        pltpu.VMEM((16400,), jnp.int32),
        pltpu.VMEM((16400,), jnp.int32),
        pltpu.VMEM((16400,), jnp.int32),
        pltpu.VMEM((2, D, SLAB), jnp.float32),
        pltpu.VMEM((RING, D), jnp.float32),
        pltpu.SemaphoreType.DMA,
        pltpu.SemaphoreType.DMA,
        pltpu.SemaphoreType.DMA,
    ],
    compiler_params=pltpu.CompilerParams(needs_layout_passes=False),
)
def _sc_gather(uid_hbm, qid_hbm, ut_hbm, qt_hbm, uout_hbm, qout_hbm,
               idv, listv, posv, bufs, stage, sA, sB, sOut):
    wid = lax.axis_index("s") * NC + lax.axis_index("c")

    s0u = wid * 61 + jnp.minimum(wid, 1)
    nsu = 61 + jnp.where(wid == 0, 1, 0)
    pltpu.sync_copy(uid_hbm, idv.at[pl.ds(0, B)])
    _phase(idv, ut_hbm, uout_hbm, listv, posv, bufs, stage,
           sA, sB, sOut, s0u, nsu, TU)

    s0q = wid * 6 + jnp.minimum(wid, 3)
    nsq = 6 + jnp.where(wid < 3, 1, 0)
    pltpu.sync_copy(qid_hbm, idv.at[pl.ds(0, B)])
    _phase(idv, qt_hbm, qout_hbm, listv, posv, bufs, stage,
           sA, sB, sOut, s0q, nsq, TQ)


BLK = 2048


def _mlp_body(u_ref, q_ref, uid_ref, qid_ref, tu_ref, tq_ref,
              w1u_ref, w1q_ref, b1_ref, w2_ref, b2_ref, o_ref):
    iot = lax.broadcasted_iota(jnp.int32, (BLK, TAIL), 1)
    uid = uid_ref[...]
    qid = qid_ref[...]
    ohu = (iot == (uid - TU)).astype(jnp.float32)
    ohq = (iot == (qid - TQ)).astype(jnp.float32)
    u_tail = jnp.dot(ohu, tu_ref[...], preferred_element_type=jnp.float32)
    q_tail = jnp.dot(ohq, tq_ref[...], preferred_element_type=jnp.float32)
    u = jnp.where(uid >= TU, u_tail, u_ref[...])
    q = jnp.where(qid >= TQ, q_tail, q_ref[...])
    h = jnp.dot(u, w1u_ref[...], preferred_element_type=jnp.float32)
    h = h + jnp.dot(q, w1q_ref[...], preferred_element_type=jnp.float32)
    h = jnp.maximum(h + b1_ref[...], 0.0)
    o = jnp.sum(h * w2_ref[...], axis=1, keepdims=True) + b2_ref[...]
    o_ref[...] = jax.nn.sigmoid(o)


def _mlp(u, q, uid, qid, tail_u, tail_q, w1u, w1q, b1, w2t, b2):
    grid = (B // BLK,)
    return pl.pallas_call(
        _mlp_body,
        grid=grid,
        in_specs=[
            pl.BlockSpec((BLK, D), lambda i: (i, 0)),
            pl.BlockSpec((BLK, D), lambda i: (i, 0)),
            pl.BlockSpec((BLK, 1), lambda i: (i, 0)),
            pl.BlockSpec((BLK, 1), lambda i: (i, 0)),
            pl.BlockSpec((TAIL, D), lambda i: (0, 0)),
            pl.BlockSpec((TAIL, D), lambda i: (0, 0)),
            pl.BlockSpec((D, H), lambda i: (0, 0)),
            pl.BlockSpec((D, H), lambda i: (0, 0)),
            pl.BlockSpec((1, H), lambda i: (0, 0)),
            pl.BlockSpec((1, H), lambda i: (0, 0)),
            pl.BlockSpec((1, 1), lambda i: (0, 0)),
        ],
        out_specs=pl.BlockSpec((BLK, 1), lambda i: (i, 0)),
        out_shape=jax.ShapeDtypeStruct((B, 1), jnp.float32),
    )(u, q, uid, qid, tail_u, tail_q, w1u, w1q, b1, w2t, b2)


def kernel(user_id, question_id, user_table, question_table, W1, b1, W2, b2):
    uid = user_id.astype(jnp.int32)
    qid = question_id.astype(jnp.int32)
    u, q = _sc_gather(uid, qid, user_table.T, question_table.T)
    tail_u = lax.slice(user_table, (TU, 0), (NU, D))
    tail_q = lax.slice(question_table, (TQ, 0), (NQ, D))
    w1u = W1[:D]
    w1q = W1[D:]
    b1r = b1.reshape(1, H)
    w2t = W2.reshape(1, H)
    b2r = b2.reshape(1, 1)
    return _mlp(u, q, uid.reshape(B, 1), qid.reshape(B, 1),
                tail_u, tail_q, w1u, w1q, b1r, w2t, b2r)


# speedup vs baseline: 2.8198x; 1.6233x over previous
import functools

import jax
import jax.numpy as jnp
from jax import lax
from jax.experimental import pallas as pl
from jax.experimental.pallas import tpu as pltpu
from jax.experimental.pallas import tpu_sc as plsc

B = 16384
D = 64
H = 64

NC, NS = 2, 16
NW = NC * NS

NU = 1000000
NQ = 100000
TAIL = 256
TU = NU - TAIL
TQ = NQ - TAIL
SLAB = 512
NS_U = 1953
NS_Q = 195
RING = 64
NSMAX = 62

_sc_mesh = plsc.VectorSubcoreMesh(
    core_axis_name="c", subcore_axis_name="s", num_cores=NC, num_subcores=NS
)


def _phase(idv, tab_hbm, out_hbm, listv, sortv, bufs, stage, hist, starts,
           cursor, sA, sB, sOut, s0, ns, tcut):
    lo_r = s0 * SLAB
    hi_r = jnp.minimum((s0 + ns) * SLAB, tcut)

    def dma_slab(t, buf_ix, sem):
        s = s0 + t
        pltpu.async_copy(
            tab_hbm.at[:, pl.ds(s * SLAB, SLAB)], bufs.at[buf_ix], sem)

    def drain_slab(buf_ix, sem):
        pltpu.make_async_copy(
            tab_hbm.at[:, pl.ds(0, SLAB)], bufs.at[buf_ix], sem).wait()

    @pl.when(ns > 0)
    def _():
        dma_slab(0, 0, sA)

    @pl.when(ns > 1)
    def _():
        dma_slab(1, 1, sB)

    def compact_body(g, cnt):
        vec = idv[pl.ds(g * 16, 16)]
        posvec = lax.iota(jnp.int32, 16) + g * 16
        m = (vec >= lo_r) & (vec < hi_r)
        packed = ((vec - lo_r) << 14) | posvec
        plsc.store_compressed(listv.at[pl.ds(cnt, 16)], packed, mask=m)
        return cnt + plsc.all_reduce_population_count(m)[0]

    cnt = lax.fori_loop(0, B // 16, compact_body, jnp.int32(0))
    nv = (cnt + 15) >> 4

    for t in range(NSMAX + 1):
        hist[t] = 0

    def hist_body(v, _):
        vec = listv[pl.ds(v * 16, 16)]
        nvalid = cnt - v * 16
        for k in range(16):
            @pl.when(k < nvalid)
            def _(k=k):
                t = vec[k] >> 23
                hist[t] = hist[t] + 1
        return 0

    lax.fori_loop(0, nv, hist_body, 0)

    run = jnp.int32(0)
    for t in range(NSMAX + 1):
        starts[t] = run
        cursor[t] = run
        run = run + hist[t]

    lane0 = lax.iota(jnp.int32, 16) == 0

    def scat_body(v, _):
        vec = listv[pl.ds(v * 16, 16)]
        nvalid = cnt - v * 16
        for k in range(16):
            @pl.when(k < nvalid)
            def _(k=k):
                val = vec[k]
                t = val >> 23
                p = cursor[t]
                cursor[t] = p + 1
                plsc.store_scatter(sortv, [jnp.full((16,), p, jnp.int32)],
                                   jnp.full((16,), val, jnp.int32),
                                   mask=lane0)
        return 0

    lax.fori_loop(0, nv, scat_body, 0)

    f0 = lax.iota(jnp.int32, 16)

    def process_slab(t, buf_ix, n):
        p16 = jnp.full((16,), buf_ix, jnp.int32)
        lo = starts[t]
        hi = starts[t + 1]

        def group_body(g, n):
            j = lo + g * 16
            vec = sortv[pl.ds(j, 16)]
            nvalid = hi - j
            for k in range(16):
                ck = k < nvalid

                @pl.when(ck & (n >= RING))
                def _():
                    pltpu.make_async_copy(
                        out_hbm.at[0], stage.at[0], sOut).wait()

                @pl.when(ck)
                def _(k=k, n=n):
                    val = vec[k]
                    slot = lax.rem(n, jnp.int32(RING))
                    o16 = jnp.full((16,), (val >> 14) & (SLAB - 1),
                                   jnp.int32)
                    for mm in range(4):
                        seg = plsc.load_gather(
                            bufs, [p16, f0 + 16 * mm, o16])
                        stage[slot, pl.ds(16 * mm, 16)] = seg
                    pltpu.async_copy(
                        stage.at[slot], out_hbm.at[val & 16383], sOut)

                n = n + jnp.where(ck, 1, 0)
            return n

        return lax.fori_loop(0, (hi - lo + 15) >> 4, group_body, n)

    def pair_body(i, n):
        t0 = 2 * i
        t1 = 2 * i + 1

        @pl.when(t0 < ns)
        def _():
            drain_slab(0, sA)

        n = lax.cond(t0 < ns, lambda n: process_slab(t0, 0, n),
                     lambda n: n, n)

        @pl.when(t0 + 2 < ns)
        def _():
            dma_slab(t0 + 2, 0, sA)

        @pl.when(t1 < ns)
        def _():
            drain_slab(1, sB)

        n = lax.cond(t1 < ns, lambda n: process_slab(t1, 1, n),
                     lambda n: n, n)

        @pl.when(t1 + 2 < ns)
        def _():
            dma_slab(t1 + 2, 1, sB)
        return n

    n = lax.fori_loop(0, (ns + 1) >> 1, pair_body, jnp.int32(0))

    def drain_out(i, _):
        pltpu.make_async_copy(out_hbm.at[0], stage.at[0], sOut).wait()
        return 0

    lax.fori_loop(0, jnp.minimum(n, RING), drain_out, 0)


@functools.partial(
    pl.kernel,
    out_type=(
        jax.ShapeDtypeStruct((B, D), jnp.float32),
        jax.ShapeDtypeStruct((B, D), jnp.float32),
    ),
    mesh=_sc_mesh,
    scratch_types=[
        pltpu.VMEM((16400,), jnp.int32),
        pltpu.VMEM((16400,), jnp.int32),
        pltpu.VMEM((16400,), jnp.int32),
        pltpu.VMEM((2, D, SLAB), jnp.float32),
        pltpu.VMEM((RING, D), jnp.float32),
        pltpu.SMEM((NSMAX + 1,), jnp.int32),
        pltpu.SMEM((NSMAX + 1,), jnp.int32),
        pltpu.SMEM((NSMAX + 1,), jnp.int32),
        pltpu.SemaphoreType.DMA,
        pltpu.SemaphoreType.DMA,
        pltpu.SemaphoreType.DMA,
    ],
    compiler_params=pltpu.CompilerParams(needs_layout_passes=False),
)
def _sc_gather(uid_hbm, qid_hbm, ut_hbm, qt_hbm, uout_hbm, qout_hbm,
               idv, listv, sortv, bufs, stage, hist, starts, cursor,
               sA, sB, sOut):
    wid = lax.axis_index("s") * NC + lax.axis_index("c")

    s0u = wid * 61 + jnp.minimum(wid, 1)
    nsu = 61 + jnp.where(wid == 0, 1, 0)
    pltpu.sync_copy(uid_hbm, idv.at[pl.ds(0, B)])
    _phase(idv, ut_hbm, uout_hbm, listv, sortv, bufs, stage, hist, starts,
           cursor, sA, sB, sOut, s0u, nsu, TU)

    s0q = wid * 6 + jnp.minimum(wid, 3)
    nsq = 6 + jnp.where(wid < 3, 1, 0)
    pltpu.sync_copy(qid_hbm, idv.at[pl.ds(0, B)])
    _phase(idv, qt_hbm, qout_hbm, listv, sortv, bufs, stage, hist, starts,
           cursor, sA, sB, sOut, s0q, nsq, TQ)


BLK = 2048


def _mlp_body(u_ref, q_ref, uid_ref, qid_ref, tu_ref, tq_ref,
              w1u_ref, w1q_ref, b1_ref, w2_ref, b2_ref, o_ref):
    iot = lax.broadcasted_iota(jnp.int32, (BLK, TAIL), 1)
    uid = uid_ref[...]
    qid = qid_ref[...]
    ohu = (iot == (uid - TU)).astype(jnp.float32)
    ohq = (iot == (qid - TQ)).astype(jnp.float32)
    u_tail = jnp.dot(ohu, tu_ref[...], preferred_element_type=jnp.float32)
    q_tail = jnp.dot(ohq, tq_ref[...], preferred_element_type=jnp.float32)
    u = jnp.where(uid >= TU, u_tail, u_ref[...])
    q = jnp.where(qid >= TQ, q_tail, q_ref[...])
    h = jnp.dot(u, w1u_ref[...], preferred_element_type=jnp.float32)
    h = h + jnp.dot(q, w1q_ref[...], preferred_element_type=jnp.float32)
    h = jnp.maximum(h + b1_ref[...], 0.0)
    o = jnp.sum(h * w2_ref[...], axis=1, keepdims=True) + b2_ref[...]
    o_ref[...] = jax.nn.sigmoid(o)


def _mlp(u, q, uid, qid, tail_u, tail_q, w1u, w1q, b1, w2t, b2):
    grid = (B // BLK,)
    return pl.pallas_call(
        _mlp_body,
        grid=grid,
        in_specs=[
            pl.BlockSpec((BLK, D), lambda i: (i, 0)),
            pl.BlockSpec((BLK, D), lambda i: (i, 0)),
            pl.BlockSpec((BLK, 1), lambda i: (i, 0)),
            pl.BlockSpec((BLK, 1), lambda i: (i, 0)),
            pl.BlockSpec((TAIL, D), lambda i: (0, 0)),
            pl.BlockSpec((TAIL, D), lambda i: (0, 0)),
            pl.BlockSpec((D, H), lambda i: (0, 0)),
            pl.BlockSpec((D, H), lambda i: (0, 0)),
            pl.BlockSpec((1, H), lambda i: (0, 0)),
            pl.BlockSpec((1, H), lambda i: (0, 0)),
            pl.BlockSpec((1, 1), lambda i: (0, 0)),
        ],
        out_specs=pl.BlockSpec((BLK, 1), lambda i: (i, 0)),
        out_shape=jax.ShapeDtypeStruct((B, 1), jnp.float32),
    )(u, q, uid, qid, tail_u, tail_q, w1u, w1q, b1, w2t, b2)


def kernel(user_id, question_id, user_table, question_table, W1, b1, W2, b2):
    uid = user_id.astype(jnp.int32)
    qid = question_id.astype(jnp.int32)
    u, q = _sc_gather(uid, qid, user_table.T, question_table.T)
    tail_u = lax.slice(user_table, (TU, 0), (NU, D))
    tail_q = lax.slice(question_table, (TQ, 0), (NQ, D))
    w1u = W1[:D]
    w1q = W1[D:]
    b1r = b1.reshape(1, H)
    w2t = W2.reshape(1, H)
    b2r = b2.reshape(1, 1)
    return _mlp(u, q, uid.reshape(B, 1), qid.reshape(B, 1),
                tail_u, tail_q, w1u, w1q, b1r, w2t, b2r)


# speedup vs baseline: 3.1359x; 1.1121x over previous
import functools

import jax
import jax.numpy as jnp
from jax import lax
from jax.experimental import pallas as pl
from jax.experimental.pallas import tpu as pltpu
from jax.experimental.pallas import tpu_sc as plsc

B = 16384
D = 64
H = 64

NC, NS = 2, 16
NW = NC * NS

NU = 1000000
NQ = 100000
TAIL = 256
TU = NU - TAIL
TQ = NQ - TAIL
SLAB = 512
NS_U = 1953
NS_Q = 195
RING = 64
NSMAX = 62

_sc_mesh = plsc.VectorSubcoreMesh(
    core_axis_name="c", subcore_axis_name="s", num_cores=NC, num_subcores=NS
)


def _phase(idv, tab_hbm, out_hbm, listv, sortv, bufs, stage, hist, starts,
           cursor, sA, sB, sOut, s0, ns, tcut):
    lo_r = s0 * SLAB
    hi_r = jnp.minimum((s0 + ns) * SLAB, tcut)

    def dma_slab(t, buf_ix, sem):
        s = s0 + t
        pltpu.async_copy(
            tab_hbm.at[:, pl.ds(s * SLAB, SLAB)], bufs.at[buf_ix], sem)

    def drain_slab(buf_ix, sem):
        pltpu.make_async_copy(
            tab_hbm.at[:, pl.ds(0, SLAB)], bufs.at[buf_ix], sem).wait()

    @pl.when(ns > 0)
    def _():
        dma_slab(0, 0, sA)

    @pl.when(ns > 1)
    def _():
        dma_slab(1, 1, sB)

    def compact_body(g, cnt):
        vec = idv[pl.ds(g * 16, 16)]
        posvec = lax.iota(jnp.int32, 16) + g * 16
        m = (vec >= lo_r) & (vec < hi_r)
        packed = ((vec - lo_r) << 14) | posvec
        plsc.store_compressed(listv.at[pl.ds(cnt, 16)], packed, mask=m)
        return cnt + plsc.all_reduce_population_count(m)[0]

    cnt = lax.fori_loop(0, B // 16, compact_body, jnp.int32(0))
    nv = (cnt + 15) >> 4

    for t in range(NSMAX + 1):
        hist[t] = 0

    def hist_body(v, _):
        vec = listv[pl.ds(v * 16, 16)]
        nvalid = cnt - v * 16
        for k in range(16):
            @pl.when(k < nvalid)
            def _(k=k):
                t = vec[k] >> 23
                hist[t] = hist[t] + 1
        return 0

    lax.fori_loop(0, nv, hist_body, 0)

    run = jnp.int32(0)
    for t in range(NSMAX + 1):
        starts[t] = run
        cursor[t] = run
        run = run + hist[t]

    lane0 = lax.iota(jnp.int32, 16) == 0

    def scat_body(v, _):
        vec = listv[pl.ds(v * 16, 16)]
        nvalid = cnt - v * 16
        for k in range(16):
            @pl.when(k < nvalid)
            def _(k=k):
                val = vec[k]
                t = val >> 23
                p = cursor[t]
                cursor[t] = p + 1
                plsc.store_scatter(sortv, [jnp.full((16,), p, jnp.int32)],
                                   jnp.full((16,), val, jnp.int32),
                                   mask=lane0)
        return 0

    lax.fori_loop(0, nv, scat_body, 0)

    f0 = lax.iota(jnp.int32, 16)

    def process_slab(t, buf_ix, n):
        p16 = jnp.full((16,), buf_ix, jnp.int32)
        lo = starts[t]
        hi = starts[t + 1]

        def group_body(g, n):
            j = lo + g * 16
            vec = sortv[pl.ds(j, 16)]
            nvalid = hi - j
            for k in range(16):
                ck = k < nvalid

                @pl.when(ck & (n >= RING))
                def _():
                    pltpu.make_async_copy(
                        out_hbm.at[0], stage.at[0], sOut).wait()

                @pl.when(ck)
                def _(k=k, n=n):
                    val = vec[k]
                    slot = lax.rem(n, jnp.int32(RING))
                    o16 = jnp.full((16,), (val >> 14) & (SLAB - 1),
                                   jnp.int32)
                    for mm in range(4):
                        seg = plsc.load_gather(
                            bufs, [p16, f0 + 16 * mm, o16])
                        stage[slot, pl.ds(16 * mm, 16)] = seg
                    pltpu.async_copy(
                        stage.at[slot], out_hbm.at[val & 16383], sOut)

                n = n + jnp.where(ck, 1, 0)
            return n

        return lax.fori_loop(0, (hi - lo + 15) >> 4, group_body, n)

    def pair_body(i, n):
        t0 = 2 * i
        t1 = 2 * i + 1

        @pl.when(t0 < ns)
        def _():
            drain_slab(0, sA)

        n = lax.cond(t0 < ns, lambda n: process_slab(t0, 0, n),
                     lambda n: n, n)

        @pl.when(t0 + 2 < ns)
        def _():
            dma_slab(t0 + 2, 0, sA)

        @pl.when(t1 < ns)
        def _():
            drain_slab(1, sB)

        n = lax.cond(t1 < ns, lambda n: process_slab(t1, 1, n),
                     lambda n: n, n)

        @pl.when(t1 + 2 < ns)
        def _():
            dma_slab(t1 + 2, 1, sB)
        return n

    n = lax.fori_loop(0, (ns + 1) >> 1, pair_body, jnp.int32(0))

    def drain_out(i, _):
        pltpu.make_async_copy(out_hbm.at[0], stage.at[0], sOut).wait()
        return 0

    lax.fori_loop(0, jnp.minimum(n, RING), drain_out, 0)


@functools.partial(
    pl.kernel,
    out_type=jax.ShapeDtypeStruct((B, D), jnp.float32),
    mesh=_sc_mesh,
    scratch_types=[
        pltpu.VMEM((16400,), jnp.int32),
        pltpu.VMEM((16400,), jnp.int32),
        pltpu.VMEM((16400,), jnp.int32),
        pltpu.VMEM((2, D, SLAB), jnp.float32),
        pltpu.VMEM((RING, D), jnp.float32),
        pltpu.SMEM((NSMAX + 1,), jnp.int32),
        pltpu.SMEM((NSMAX + 1,), jnp.int32),
        pltpu.SMEM((NSMAX + 1,), jnp.int32),
        pltpu.SemaphoreType.DMA,
        pltpu.SemaphoreType.DMA,
        pltpu.SemaphoreType.DMA,
    ],
    compiler_params=pltpu.CompilerParams(needs_layout_passes=False),
)
def _sc_gather(uid_hbm, ut_hbm, uout_hbm,
               idv, listv, sortv, bufs, stage, hist, starts, cursor,
               sA, sB, sOut):
    wid = lax.axis_index("s") * NC + lax.axis_index("c")

    s0u = wid * 61 + jnp.minimum(wid, 1)
    nsu = 61 + jnp.where(wid == 0, 1, 0)
    pltpu.sync_copy(uid_hbm, idv.at[pl.ds(0, B)])
    _phase(idv, ut_hbm, uout_hbm, listv, sortv, bufs, stage, hist, starts,
           cursor, sA, sB, sOut, s0u, nsu, TU)


BPW = B // NW


@functools.partial(
    pl.kernel,
    out_type=jax.ShapeDtypeStruct((B, D), jnp.float32),
    mesh=_sc_mesh,
    scratch_types=[
        pltpu.VMEM((BPW,), jnp.int32),
        pltpu.VMEM((BPW, D), jnp.float32),
        pltpu.SemaphoreType.DMA,
    ],
    compiler_params=pltpu.CompilerParams(needs_layout_passes=False),
)
def _sc_qgather(qid_hbm, qt_hbm, qout_hbm, qidx_v, qrows_v, sem):
    wid = lax.axis_index("s") * NC + lax.axis_index("c")
    base = wid * BPW
    pltpu.sync_copy(qid_hbm.at[pl.ds(base, BPW)], qidx_v)

    def body(g, _):
        qvec = qidx_v[pl.ds(g * 16, 16)]
        for k in range(16):
            pltpu.async_copy(qt_hbm.at[qvec[k]], qrows_v.at[g * 16 + k],
                             sem)
        return 0

    lax.fori_loop(0, BPW // 16, body, 0)
    pltpu.make_async_copy(qt_hbm.at[pl.ds(0, BPW)], qrows_v, sem).wait()
    pltpu.sync_copy(qrows_v, qout_hbm.at[pl.ds(base, BPW)])


BLK = 2048


def _mlp_body(u_ref, q_ref, uid_ref, tu_ref,
              w1u_ref, w1q_ref, b1_ref, w2_ref, b2_ref, o_ref):
    iot = lax.broadcasted_iota(jnp.int32, (BLK, TAIL), 1)
    uid = uid_ref[...]
    ohu = (iot == (uid - TU)).astype(jnp.float32)
    u_tail = jnp.dot(ohu, tu_ref[...], preferred_element_type=jnp.float32)
    u = jnp.where(uid >= TU, u_tail, u_ref[...])
    q = q_ref[...]
    h = jnp.dot(u, w1u_ref[...], preferred_element_type=jnp.float32)
    h = h + jnp.dot(q, w1q_ref[...], preferred_element_type=jnp.float32)
    h = jnp.maximum(h + b1_ref[...], 0.0)
    o = jnp.sum(h * w2_ref[...], axis=1, keepdims=True) + b2_ref[...]
    o_ref[...] = jax.nn.sigmoid(o)


def _mlp(u, q, uid, tail_u, w1u, w1q, b1, w2t, b2):
    grid = (B // BLK,)
    return pl.pallas_call(
        _mlp_body,
        grid=grid,
        in_specs=[
            pl.BlockSpec((BLK, D), lambda i: (i, 0)),
            pl.BlockSpec((BLK, D), lambda i: (i, 0)),
            pl.BlockSpec((BLK, 1), lambda i: (i, 0)),
            pl.BlockSpec((TAIL, D), lambda i: (0, 0)),
            pl.BlockSpec((D, H), lambda i: (0, 0)),
            pl.BlockSpec((D, H), lambda i: (0, 0)),
            pl.BlockSpec((1, H), lambda i: (0, 0)),
            pl.BlockSpec((1, H), lambda i: (0, 0)),
            pl.BlockSpec((1, 1), lambda i: (0, 0)),
        ],
        out_specs=pl.BlockSpec((BLK, 1), lambda i: (i, 0)),
        out_shape=jax.ShapeDtypeStruct((B, 1), jnp.float32),
    )(u, q, uid, tail_u, w1u, w1q, b1, w2t, b2)


def kernel(user_id, question_id, user_table, question_table, W1, b1, W2, b2):
    uid = user_id.astype(jnp.int32)
    qid = question_id.astype(jnp.int32)
    u = _sc_gather(uid, user_table.T)
    q = _sc_qgather(qid, question_table)
    tail_u = lax.slice(user_table, (TU, 0), (NU, D))
    w1u = W1[:D]
    w1q = W1[D:]
    b1r = b1.reshape(1, H)
    w2t = W2.reshape(1, H)
    b2r = b2.reshape(1, 1)
    return _mlp(u, q, uid.reshape(B, 1), tail_u, w1u, w1q, b1r, w2t, b2r)
